# baseline (speedup 1.0000x reference)
import functools

import jax
import jax.numpy as jnp
from jax import lax
from jax.experimental import pallas as pl
from jax.experimental.pallas import tpu as pltpu
from jax.experimental.pallas import tpu_sc as plsc

L = 16


@functools.lru_cache(maxsize=None)
def _build_sc_kernel(B, C, V, D):
    info = plsc.get_sparse_core_info()
    NC, NS = info.num_cores, info.num_subcores
    NW = NC * NS
    assert B % (NW * L) == 0
    BPW = B // NW
    CB = L
    NCH = BPW // CB
    DB = 8
    NDB = D // DB

    mesh = plsc.VectorSubcoreMesh(core_axis_name="c", subcore_axis_name="s")

    @functools.partial(
        pl.kernel,
        mesh=mesh,
        out_type=jax.ShapeDtypeStruct((B * C,), jnp.float32),
        compiler_params=pltpu.CompilerParams(
            needs_layout_passes=False,
            use_tc_tiling_on_sc=False,
        ),
        scratch_types=[
            pltpu.VMEM((CB,), jnp.int32),
            pltpu.VMEM((CB * C,), jnp.int32),
            pltpu.VMEM((CB, D), jnp.float32),
            pltpu.VMEM((CB * C, D), jnp.float32),
            pltpu.VMEM((CB * C,), jnp.float32),
            pltpu.SemaphoreType.DMA,
            pltpu.SemaphoreType.DMA,
        ],
    )
    def sc_kernel(center_hbm, ctx_hbm, wc_hbm, wk_hbm, out_hbm,
                  cidx, kidx, crows, krows, outv, sem1, sem2):
        wid = lax.axis_index("s") * NC + lax.axis_index("c")
        wbase = wid * BPW
        iota = lax.broadcasted_iota(jnp.int32, (L,), 0)
        zerov = jnp.zeros((L,), jnp.float32)

        def chunk_body(i, carry):
            base = wbase + i * CB
            pltpu.sync_copy(center_hbm.at[pl.ds(base, CB)], cidx)
            pltpu.sync_copy(ctx_hbm.at[pl.ds(base * C, CB * C)], kidx)
            h1 = pltpu.async_copy(wc_hbm.at[cidx], crows, sem1)
            h2 = pltpu.async_copy(wk_hbm.at[kidx], krows, sem2)
            h1.wait()
            h2.wait()

            def dblk_body(dblk, accs):
                d0 = dblk * DB
                ccols = [
                    plsc.load_gather(crows, [iota, iota * 0 + (d0 + d)])
                    for d in range(DB)
                ]
                new_accs = []
                for c in range(C):
                    a = accs[c]
                    rowc = iota * C + c
                    for d in range(DB):
                        kcol = plsc.load_gather(krows, [rowc, iota * 0 + (d0 + d)])
                        a = a + ccols[d] * kcol
                    new_accs.append(a)
                return tuple(new_accs)

            accs = lax.fori_loop(0, NDB, dblk_body, (zerov,) * C)
            for c in range(C):
                plsc.store_scatter(outv, [iota * C + c], accs[c])
            pltpu.sync_copy(outv, out_hbm.at[pl.ds(base * C, CB * C)])
            return carry

        lax.fori_loop(0, NCH, chunk_body, 0)

    return sc_kernel


def kernel(center, context, W_center, W_context):
    B, C = context.shape
    V, D = W_center.shape
    center = jnp.asarray(center, jnp.int32)
    ctx_flat = jnp.asarray(context, jnp.int32).reshape(B * C)
    sc = _build_sc_kernel(B, C, V, D)
    out_flat = sc(center, ctx_flat, W_center, W_context)
    return out_flat.reshape(B, C)
